# Initial kernel scaffold; baseline (speedup 1.0000x reference)
import jax
import jax.numpy as jnp
from jax.experimental import pallas as pl


def kernel(token_logits, sampling_params):
    raise NotImplementedError("write your pallas kernel here")


# speedup vs baseline: 10.4805x; 10.4805x over previous
import functools

import jax
import jax.numpy as jnp
from jax import lax
from jax.experimental import pallas as pl
from jax.experimental.pallas import tpu as pltpu
from jax.experimental.pallas import tpu_sc as plsc

NC, NS, L = 2, 16, 16
NW = NC * NS
B, V = 64, 100000
ROWS_PER_W = B // NW
NVREG = V // L
CAND = 256
NEG = -3.4028235e38
IGNORED = -3000.0


def _sc_body(logits_hbm, out_vals_hbm, out_idx_hbm, row_v, hist_v,
             cvals_v, cidx_v):
    wid = lax.axis_index("s") * NC + lax.axis_index("c")
    lane = lax.broadcasted_iota(jnp.int32, (L,), 0)
    ones = jnp.ones((L,), jnp.int32)

    def uvec(i):
        v = row_v[pl.ds(i * L, L)]
        b = plsc.bitcast(v, jnp.int32)
        s = b >> 31
        u = b ^ (s & jnp.int32(0x7FFFFFFF))
        return v, u

    def hist_pass(dshift, pshift, pval, rank):
        def zbody(k, _):
            hist_v[pl.ds(k * L, L)] = jnp.zeros((L,), jnp.int32)
            return 0
        lax.fori_loop(0, 4096 // L, zbody, 0)

        def hbody(i, _):
            _, u = uvec(i)
            if dshift == 24:
                d = (u >> 24) + 128
            else:
                d = (u >> dshift) & 0xFF
            slot = d * L + lane
            if pshift is None:
                plsc.addupdate_scatter(hist_v, [slot], ones)
            else:
                m = (u >> pshift) == pval
                plsc.addupdate_scatter(hist_v, [slot], ones, mask=m)
            return 0
        lax.fori_loop(0, NVREG, hbody, 0)

        def sbody(k, carry):
            acc, found, bsel, rsel = carry
            bk = 255 - k
            cnt = jnp.sum(hist_v[pl.ds(bk * L, L)])
            hit = jnp.logical_and(found == 0, acc + cnt >= rank)
            bsel = jnp.where(hit, bk, bsel)
            rsel = jnp.where(hit, rank - acc, rsel)
            found = jnp.where(hit, 1, found)
            acc = jnp.where(found == 0, acc + cnt, acc)
            return acc, found, bsel, rsel
        _, _, bsel, rsel = lax.fori_loop(
            0, 256, sbody,
            (jnp.int32(0), jnp.int32(0), jnp.int32(0), jnp.int32(1)))
        return bsel, rsel

    for rr in range(ROWS_PER_W):
        row = wid * ROWS_PER_W + rr
        pltpu.sync_copy(logits_hbm.at[row], row_v)

        b0, r1 = hist_pass(24, None, None, jnp.int32(50))
        d0s = b0 - 128
        b1, r2 = hist_pass(16, 24, d0s, r1)
        pre16 = (d0s << 8) | b1
        b2, _ = hist_pass(8, 16, pre16, r2)
        pre24 = (pre16 << 8) | b2
        thresh = pre24 << 8

        def ibody(k, _):
            cvals_v[pl.ds(k * L, L)] = jnp.full((L,), NEG, jnp.float32)
            cidx_v[pl.ds(k * L, L)] = jnp.zeros((L,), jnp.int32)
            return 0
        lax.fori_loop(0, CAND // L, ibody, 0)

        def cbody(i, ptr):
            v, u = uvec(i)
            m = u >= thresh
            ptrc = jnp.minimum(ptr, CAND - L)
            mw = jnp.logical_and(m, ptr <= CAND - L)
            plsc.store_compressed(cvals_v.at[pl.ds(ptrc, L)], v, mask=mw)
            plsc.store_compressed(cidx_v.at[pl.ds(ptrc, L)], i * L + lane,
                                  mask=mw)
            cnt = jnp.max(plsc.all_reduce_population_count(mw))
            return ptr + cnt
        lax.fori_loop(0, NVREG, cbody, jnp.int32(0))

        pltpu.sync_copy(cvals_v, out_vals_hbm.at[row])
        pltpu.sync_copy(cidx_v, out_idx_hbm.at[row])


_sc_select = pl.kernel(
    _sc_body,
    out_type=[jax.ShapeDtypeStruct((B, CAND), jnp.float32),
              jax.ShapeDtypeStruct((B, CAND), jnp.int32)],
    mesh=plsc.VectorSubcoreMesh(core_axis_name="c", subcore_axis_name="s",
                                num_cores=NC, num_subcores=NS),
    scratch_types=[pltpu.VMEM((V,), jnp.float32),
                   pltpu.VMEM((4096,), jnp.int32),
                   pltpu.VMEM((CAND,), jnp.float32),
                   pltpu.VMEM((CAND,), jnp.int32)],
    compiler_params=pltpu.CompilerParams(needs_layout_passes=False),
)


def _tc_body(vals_ref, idx_ref, tk_ref, tp_ref, tt_ref, out_ref):
    K = 50
    vals = vals_ref[:]
    idxf = idx_ref[:].astype(jnp.float32)

    sv_cols = []
    si_cols = []
    for _ in range(K):
        m = jnp.max(vals, axis=1, keepdims=True)
        is_m = vals == m
        isel = jnp.min(jnp.where(is_m, idxf, jnp.float32(1e9)), axis=1,
                       keepdims=True)
        sv_cols.append(m)
        si_cols.append(isel)
        vals = jnp.where(jnp.logical_and(is_m, idxf == isel), NEG, vals)
    sv = jnp.concatenate(sv_cols, axis=1)
    si = jnp.concatenate(si_cols, axis=1)

    tk = tk_ref[:]
    tp = tp_ref[:]
    tt = tt_ref[:]

    posf = lax.broadcasted_iota(jnp.int32, (B, K), 1).astype(jnp.float32)
    s1 = jnp.where(posf >= tk, IGNORED, sv) / tt

    m1 = jnp.max(s1, axis=1, keepdims=True)
    e1 = jnp.exp(s1 - m1)
    p1 = e1 / jnp.sum(e1, axis=1, keepdims=True)

    rr = lax.broadcasted_iota(jnp.int32, (K, K), 0)
    cc = lax.broadcasted_iota(jnp.int32, (K, K), 1)
    U = (rr <= cc).astype(jnp.float32)
    c1 = lax.dot_general(p1, U, (((1,), (0,)), ((), ())),
                         precision=lax.Precision.HIGHEST,
                         preferred_element_type=jnp.float32)

    gmin = jnp.min(p1[:, 0:1])
    tpe = jnp.maximum(gmin, tp)
    mask2 = jnp.logical_and(c1 > tpe, posf >= 1.0)
    s2 = jnp.where(mask2, IGNORED, s1)

    m2 = jnp.max(s2, axis=1, keepdims=True)
    e2 = jnp.exp(s2 - m2)
    p2 = e2 / jnp.sum(e2, axis=1, keepdims=True)
    c2 = lax.dot_general(p2, U, (((1,), (0,)), ((), ())),
                         precision=lax.Precision.HIGHEST,
                         preferred_element_type=jnp.float32)

    cnt = jnp.sum((c2 < 0.5).astype(jnp.int32), axis=1, keepdims=True)
    oh = lax.broadcasted_iota(jnp.int32, (B, K), 1) == cnt
    token = jnp.sum(jnp.where(oh, si, 0.0), axis=1, keepdims=True)
    out_ref[:] = token.astype(jnp.int32)


_tc_finish = pl.pallas_call(
    _tc_body,
    out_shape=jax.ShapeDtypeStruct((B, 1), jnp.int32),
)


@jax.jit
def kernel(token_logits, sampling_params):
    cv, ci = _sc_select(token_logits)
    tk = sampling_params[:, 0:1]
    tp = sampling_params[:, 1:2]
    tt = sampling_params[:, 2:3]
    return _tc_finish(cv, ci, tk, tp, tt).reshape(-1)


# speedup vs baseline: 17.1838x; 1.6396x over previous
import functools

import jax
import jax.numpy as jnp
from jax import lax
from jax.experimental import pallas as pl
from jax.experimental.pallas import tpu as pltpu
from jax.experimental.pallas import tpu_sc as plsc

NC, NS, L = 2, 16, 16
NW = NC * NS
B, V = 64, 100000
ROWS_PER_W = B // NW
NVREG = V // L
CAND = 256
NEG = -3.4028235e38
IGNORED = -3000.0


CAND1 = 4096
NB = 1024
UNROLL = 10
NBLK = NVREG // UNROLL


def _monotone(b):
    s = b >> 31
    return b ^ (s & 0x7FFFFFFF)


def _sc_body(logits_hbm, out_vals_hbm, out_idx_hbm, row_v, hist_v,
             cv1_v, ci1_v, cv2_v, ci2_v):
    wid = lax.axis_index("s") * NC + lax.axis_index("c")
    lane = lax.broadcasted_iota(jnp.int32, (L,), 0)
    ones = jnp.ones((L,), jnp.int32)
    lanebase = lane * NB + 512

    def refine_pass(dshift, pshift, pval, rank):
        def zbody(k, _):
            hist_v[pl.ds(k * L, L)] = jnp.zeros((L,), jnp.int32)
            return 0
        lax.fori_loop(0, 4096 // L, zbody, 0)

        def hbody(i, _):
            u = _monotone(plsc.bitcast(cv1_v[pl.ds(i * L, L)], jnp.int32))
            if dshift == 24:
                d = (u >> 24) + 128
            else:
                d = (u >> dshift) & 0xFF
            slot = d * L + lane
            if pshift is None:
                plsc.addupdate_scatter(hist_v, [slot], ones)
            else:
                m = (u >> pshift) == pval
                plsc.addupdate_scatter(hist_v, [slot], ones, mask=m)
            return 0
        lax.fori_loop(0, CAND1 // L, hbody, 0)

        def sbody(k, carry):
            acc, found, bsel, rsel = carry
            bk = 255 - k
            cnt = jnp.sum(hist_v[pl.ds(bk * L, L)])
            hit = jnp.logical_and(found == 0, acc + cnt >= rank)
            bsel = jnp.where(hit, bk, bsel)
            rsel = jnp.where(hit, rank - acc, rsel)
            found = jnp.where(hit, 1, found)
            acc = jnp.where(found == 0, acc + cnt, acc)
            return acc, found, bsel, rsel
        _, _, bsel, rsel = lax.fori_loop(
            0, 256, sbody,
            (jnp.int32(0), jnp.int32(0), jnp.int32(0), jnp.int32(1)))
        return bsel, rsel

    for rr in range(ROWS_PER_W):
        row = wid * ROWS_PER_W + rr
        pltpu.sync_copy(logits_hbm.at[row], row_v)

        def zcbody(k, _):
            base = k * (8 * L)
            for j in range(8):
                hist_v[pl.ds(base + j * L, L)] = jnp.zeros((L,), jnp.int32)
            return 0
        lax.fori_loop(0, NB * L // (8 * L), zcbody, 0)

        def chbody(ii, _):
            base = ii * (UNROLL * L)
            for k in range(UNROLL):
                u = _monotone(plsc.bitcast(row_v[pl.ds(base + k * L, L)],
                                           jnp.int32))
                plsc.addupdate_scatter(hist_v, [lanebase + (u >> 22)], ones)
            return 0
        lax.fori_loop(0, NBLK, chbody, 0)

        def gbody(a, carry):
            acc, found, gsel, accsel, tvsel = carry
            g = 63 - a
            tv = hist_v[pl.ds(g * L, L)]
            for l in range(1, L):
                tv = tv + hist_v[pl.ds(l * NB + g * L, L)]
            gsum = jnp.sum(tv)
            hit = jnp.logical_and(found == 0, acc + gsum >= 50)
            gsel = jnp.where(hit, g, gsel)
            accsel = jnp.where(hit, acc, accsel)
            tvsel = jnp.where(hit, tv, tvsel)
            found = jnp.where(hit, 1, found)
            acc = jnp.where(found == 0, acc + gsum, acc)
            return acc, found, gsel, accsel, tvsel
        _, _, gsel, accsel, tvsel = lax.fori_loop(
            0, NB // L, gbody,
            (jnp.int32(0), jnp.int32(0), jnp.int32(0), jnp.int32(0),
             jnp.zeros((L,), jnp.int32)))
        rtv = lax.rev(tvsel, (0,))
        inc = plsc.cumsum(rtv)
        crossed = (accsel + inc) >= 50
        i0 = jnp.max(plsc.all_reduce_ffs(crossed))
        b10 = gsel * L + (15 - i0)
        thresh1 = (b10 - 512) << 22

        def icbody(k, _):
            base = k * (8 * L)
            for j in range(8):
                cv1_v[pl.ds(base + j * L, L)] = jnp.full((L,), NEG,
                                                         jnp.float32)
            return 0
        lax.fori_loop(0, CAND1 // (8 * L), icbody, 0)

        def c1body(ii, ptr):
            base = ii * (UNROLL * L)
            for k in range(UNROLL):
                off = base + k * L
                v = row_v[pl.ds(off, L)]
                u = _monotone(plsc.bitcast(v, jnp.int32))
                m = u >= thresh1
                ptrc = jnp.minimum(ptr, CAND1 - L)
                mw = jnp.logical_and(m, ptr <= CAND1 - L)
                plsc.store_compressed(cv1_v.at[pl.ds(ptrc, L)], v, mask=mw)
                plsc.store_compressed(ci1_v.at[pl.ds(ptrc, L)], off + lane,
                                      mask=mw)
                ptr = ptr + jnp.max(plsc.all_reduce_population_count(mw))
            return ptr
        lax.fori_loop(0, NBLK, c1body, jnp.int32(0))

        b0, r1 = refine_pass(24, None, None, jnp.int32(50))
        d0s = b0 - 128
        b1, r2 = refine_pass(16, 24, d0s, r1)
        pre16 = (d0s << 8) | b1
        b2, _ = refine_pass(8, 16, pre16, r2)
        pre24 = (pre16 << 8) | b2
        thresh2 = pre24 << 8

        def i2body(k, _):
            cv2_v[pl.ds(k * L, L)] = jnp.full((L,), NEG, jnp.float32)
            cidx0 = jnp.zeros((L,), jnp.int32)
            ci2_v[pl.ds(k * L, L)] = cidx0
            return 0
        lax.fori_loop(0, CAND // L, i2body, 0)

        def c2body(i, ptr):
            v = cv1_v[pl.ds(i * L, L)]
            ivec = ci1_v[pl.ds(i * L, L)]
            u = _monotone(plsc.bitcast(v, jnp.int32))
            m = u >= thresh2
            ptrc = jnp.minimum(ptr, CAND - L)
            mw = jnp.logical_and(m, ptr <= CAND - L)
            plsc.store_compressed(cv2_v.at[pl.ds(ptrc, L)], v, mask=mw)
            plsc.store_compressed(ci2_v.at[pl.ds(ptrc, L)], ivec, mask=mw)
            return ptr + jnp.max(plsc.all_reduce_population_count(mw))
        lax.fori_loop(0, CAND1 // L, c2body, jnp.int32(0))

        pltpu.sync_copy(cv2_v, out_vals_hbm.at[row])
        pltpu.sync_copy(ci2_v, out_idx_hbm.at[row])


_sc_select = pl.kernel(
    _sc_body,
    out_type=[jax.ShapeDtypeStruct((B, CAND), jnp.float32),
              jax.ShapeDtypeStruct((B, CAND), jnp.int32)],
    mesh=plsc.VectorSubcoreMesh(core_axis_name="c", subcore_axis_name="s",
                                num_cores=NC, num_subcores=NS),
    scratch_types=[pltpu.VMEM((V,), jnp.float32),
                   pltpu.VMEM((L * NB,), jnp.int32),
                   pltpu.VMEM((CAND1,), jnp.float32),
                   pltpu.VMEM((CAND1,), jnp.int32),
                   pltpu.VMEM((CAND,), jnp.float32),
                   pltpu.VMEM((CAND,), jnp.int32)],
    compiler_params=pltpu.CompilerParams(needs_layout_passes=False),
)


def _tc_body(vals_ref, idx_ref, tk_ref, tp_ref, tt_ref, out_ref):
    K = 50
    vals = vals_ref[:]
    idxf = idx_ref[:].astype(jnp.float32)

    sv_cols = []
    si_cols = []
    for _ in range(K):
        m = jnp.max(vals, axis=1, keepdims=True)
        is_m = vals == m
        isel = jnp.min(jnp.where(is_m, idxf, jnp.float32(1e9)), axis=1,
                       keepdims=True)
        sv_cols.append(m)
        si_cols.append(isel)
        vals = jnp.where(jnp.logical_and(is_m, idxf == isel), NEG, vals)
    sv = jnp.concatenate(sv_cols, axis=1)
    si = jnp.concatenate(si_cols, axis=1)

    tk = tk_ref[:]
    tp = tp_ref[:]
    tt = tt_ref[:]

    posf = lax.broadcasted_iota(jnp.int32, (B, K), 1).astype(jnp.float32)
    s1 = jnp.where(posf >= tk, IGNORED, sv) / tt

    m1 = jnp.max(s1, axis=1, keepdims=True)
    e1 = jnp.exp(s1 - m1)
    p1 = e1 / jnp.sum(e1, axis=1, keepdims=True)

    rr = lax.broadcasted_iota(jnp.int32, (K, K), 0)
    cc = lax.broadcasted_iota(jnp.int32, (K, K), 1)
    U = (rr <= cc).astype(jnp.float32)
    c1 = lax.dot_general(p1, U, (((1,), (0,)), ((), ())),
                         precision=lax.Precision.HIGHEST,
                         preferred_element_type=jnp.float32)

    gmin = jnp.min(p1[:, 0:1])
    tpe = jnp.maximum(gmin, tp)
    mask2 = jnp.logical_and(c1 > tpe, posf >= 1.0)
    s2 = jnp.where(mask2, IGNORED, s1)

    m2 = jnp.max(s2, axis=1, keepdims=True)
    e2 = jnp.exp(s2 - m2)
    p2 = e2 / jnp.sum(e2, axis=1, keepdims=True)
    c2 = lax.dot_general(p2, U, (((1,), (0,)), ((), ())),
                         precision=lax.Precision.HIGHEST,
                         preferred_element_type=jnp.float32)

    cnt = jnp.sum((c2 < 0.5).astype(jnp.int32), axis=1, keepdims=True)
    oh = lax.broadcasted_iota(jnp.int32, (B, K), 1) == cnt
    token = jnp.sum(jnp.where(oh, si, 0.0), axis=1, keepdims=True)
    out_ref[:] = token.astype(jnp.int32)


_tc_finish = pl.pallas_call(
    _tc_body,
    out_shape=jax.ShapeDtypeStruct((B, 1), jnp.int32),
)


@jax.jit
def kernel(token_logits, sampling_params):
    cv, ci = _sc_select(token_logits)
    tk = sampling_params[:, 0:1]
    tp = sampling_params[:, 1:2]
    tt = sampling_params[:, 2:3]
    return _tc_finish(cv, ci, tk, tp, tt).reshape(-1)


# speedup vs baseline: 64.9132x; 3.7776x over previous
import functools

import jax
import jax.numpy as jnp
from jax import lax
from jax.experimental import pallas as pl
from jax.experimental.pallas import tpu as pltpu
from jax.experimental.pallas import tpu_sc as plsc

NC, NS, L = 2, 16, 16
NW = NC * NS
B, V = 64, 100000
ROWS_PER_W = B // NW
NVREG = V // L
CAND = 256
NEG = -3.4028235e38
IGNORED = -3000.0


CAND1 = 4096
NB = 1024
UNROLL = 10
NBLK = NVREG // UNROLL


def _monotone(b):
    s = b >> 31
    return b ^ (s & 0x7FFFFFFF)


def _sc_body(logits_hbm, out_vals_hbm, out_idx_hbm, row_v, hist_v,
             cv1_v, ci1_v, cv2_v, ci2_v):
    wid = lax.axis_index("s") * NC + lax.axis_index("c")
    lane = lax.broadcasted_iota(jnp.int32, (L,), 0)
    ones = jnp.ones((L,), jnp.int32)
    zeros16 = jnp.zeros((L,), jnp.int32)
    negs16 = jnp.full((L,), NEG, jnp.float32)
    lanebase_c = lane * NB + 512
    lanebase_r = lane * 256

    def scan_hist(nb, rank):
        ngr = nb // L

        @plsc.parallel_loop(0, ngr, carry=(jnp.int32(0), jnp.int32(0),
                                           jnp.int32(0), jnp.int32(0),
                                           zeros16))
        def gscan(a, carry):
            acc, found, gsel, accsel, tvsel = carry
            g = (ngr - 1) - a
            tv = hist_v[pl.ds(g * L, L)]
            for l in range(1, L):
                tv = tv + hist_v[pl.ds(l * nb + g * L, L)]
            gsum = jnp.sum(tv)
            hit = jnp.logical_and(found == 0, acc + gsum >= rank)
            gsel = jnp.where(hit, g, gsel)
            accsel = jnp.where(hit, acc, accsel)
            tvsel = jnp.where(hit, tv, tvsel)
            found = jnp.where(hit, 1, found)
            acc = jnp.where(found == 0, acc + gsum, acc)
            return acc, found, gsel, accsel, tvsel
        _, _, gsel, accsel, tvsel = gscan
        rtv = lax.rev(tvsel, (0,))
        inc = plsc.cumsum(rtv)
        crossed = (accsel + inc) >= rank
        i0 = jnp.max(plsc.all_reduce_ffs(crossed))
        bucket = gsel * L + (15 - i0)
        above = accsel + jnp.sum(jnp.where(lane == i0, inc - rtv, 0))
        return bucket, rank - above

    def refine_pass(dshift, pshift, pval, rank):
        @plsc.parallel_loop(0, 4096 // L, unroll=8)
        def _(k):
            hist_v[pl.ds(k * L, L)] = zeros16

        @plsc.parallel_loop(0, CAND1 // L, unroll=8)
        def _(i):
            u = _monotone(plsc.bitcast(cv1_v[pl.ds(i * L, L)], jnp.int32))
            if dshift == 24:
                d = (u >> 24) + 128
            else:
                d = (u >> dshift) & 0xFF
            slot = lanebase_r + d
            if pshift is None:
                plsc.addupdate_scatter(hist_v, [slot], ones)
            else:
                m = (u >> pshift) == pval
                plsc.addupdate_scatter(hist_v, [slot], ones, mask=m)

        return scan_hist(256, rank)

    for rr in range(ROWS_PER_W):
        row = wid * ROWS_PER_W + rr
        pltpu.sync_copy(logits_hbm.at[row], row_v)

        @plsc.parallel_loop(0, NB * L // L, unroll=8)
        def _(k):
            hist_v[pl.ds(k * L, L)] = zeros16

        @plsc.parallel_loop(0, NVREG, unroll=UNROLL)
        def _(i):
            u = _monotone(plsc.bitcast(row_v[pl.ds(i * L, L)], jnp.int32))
            plsc.addupdate_scatter(hist_v, [lanebase_c + (u >> 22)], ones)

        b10, _r = scan_hist(NB, jnp.int32(50))
        thresh1 = (b10 - 512) << 22

        @plsc.parallel_loop(0, CAND1 // L, unroll=8)
        def _(k):
            cv1_v[pl.ds(k * L, L)] = negs16

        @plsc.parallel_loop(0, NVREG, unroll=UNROLL, carry=zeros16)
        def c1ptr(i, ptr):
            v = row_v[pl.ds(i * L, L)]
            u = _monotone(plsc.bitcast(v, jnp.int32))
            m = u >= thresh1
            mi = m.astype(jnp.int32)
            offs = ptr + (plsc.cumsum(mi) - mi)
            mw = jnp.logical_and(m, offs < CAND1)
            plsc.store_scatter(cv1_v, [offs], v, mask=mw)
            plsc.store_scatter(ci1_v, [offs], i * L + lane, mask=mw)
            return ptr + plsc.all_reduce_population_count(m)
        del c1ptr

        b0, r1 = refine_pass(24, None, None, jnp.int32(50))
        d0s = b0 - 128
        b1, r2 = refine_pass(16, 24, d0s, r1)
        pre16 = (d0s << 8) | b1
        b2, _ = refine_pass(8, 16, pre16, r2)
        pre24 = (pre16 << 8) | b2
        thresh2 = pre24 << 8

        @plsc.parallel_loop(0, CAND // L, unroll=4)
        def _(k):
            cv2_v[pl.ds(k * L, L)] = negs16
            ci2_v[pl.ds(k * L, L)] = zeros16

        @plsc.parallel_loop(0, CAND1 // L, unroll=8, carry=zeros16)
        def c2ptr(i, ptr):
            v = cv1_v[pl.ds(i * L, L)]
            ivec = ci1_v[pl.ds(i * L, L)]
            u = _monotone(plsc.bitcast(v, jnp.int32))
            m = u >= thresh2
            mi = m.astype(jnp.int32)
            offs = ptr + (plsc.cumsum(mi) - mi)
            mw = jnp.logical_and(m, offs < CAND)
            plsc.store_scatter(cv2_v, [offs], v, mask=mw)
            plsc.store_scatter(ci2_v, [offs], ivec, mask=mw)
            return ptr + plsc.all_reduce_population_count(m)
        del c2ptr

        pltpu.sync_copy(cv2_v, out_vals_hbm.at[row])
        pltpu.sync_copy(ci2_v, out_idx_hbm.at[row])


_sc_select = pl.kernel(
    _sc_body,
    out_type=[jax.ShapeDtypeStruct((B, CAND), jnp.float32),
              jax.ShapeDtypeStruct((B, CAND), jnp.int32)],
    mesh=plsc.VectorSubcoreMesh(core_axis_name="c", subcore_axis_name="s",
                                num_cores=NC, num_subcores=NS),
    scratch_types=[pltpu.VMEM((V,), jnp.float32),
                   pltpu.VMEM((L * NB,), jnp.int32),
                   pltpu.VMEM((CAND1,), jnp.float32),
                   pltpu.VMEM((CAND1,), jnp.int32),
                   pltpu.VMEM((CAND,), jnp.float32),
                   pltpu.VMEM((CAND,), jnp.int32)],
    compiler_params=pltpu.CompilerParams(needs_layout_passes=False),
)


def _tc_body(vals_ref, idx_ref, tk_ref, tp_ref, tt_ref, out_ref):
    K = 50
    vals = vals_ref[:]
    idxf = idx_ref[:].astype(jnp.float32)

    sv_cols = []
    si_cols = []
    for _ in range(K):
        m = jnp.max(vals, axis=1, keepdims=True)
        is_m = vals == m
        isel = jnp.min(jnp.where(is_m, idxf, jnp.float32(1e9)), axis=1,
                       keepdims=True)
        sv_cols.append(m)
        si_cols.append(isel)
        vals = jnp.where(jnp.logical_and(is_m, idxf == isel), NEG, vals)
    sv = jnp.concatenate(sv_cols, axis=1)
    si = jnp.concatenate(si_cols, axis=1)

    tk = tk_ref[:]
    tp = tp_ref[:]
    tt = tt_ref[:]

    posf = lax.broadcasted_iota(jnp.int32, (B, K), 1).astype(jnp.float32)
    s1 = jnp.where(posf >= tk, IGNORED, sv) / tt

    m1 = jnp.max(s1, axis=1, keepdims=True)
    e1 = jnp.exp(s1 - m1)
    p1 = e1 / jnp.sum(e1, axis=1, keepdims=True)

    rr = lax.broadcasted_iota(jnp.int32, (K, K), 0)
    cc = lax.broadcasted_iota(jnp.int32, (K, K), 1)
    U = (rr <= cc).astype(jnp.float32)
    c1 = lax.dot_general(p1, U, (((1,), (0,)), ((), ())),
                         precision=lax.Precision.HIGHEST,
                         preferred_element_type=jnp.float32)

    gmin = jnp.min(p1[:, 0:1])
    tpe = jnp.maximum(gmin, tp)
    mask2 = jnp.logical_and(c1 > tpe, posf >= 1.0)
    s2 = jnp.where(mask2, IGNORED, s1)

    m2 = jnp.max(s2, axis=1, keepdims=True)
    e2 = jnp.exp(s2 - m2)
    p2 = e2 / jnp.sum(e2, axis=1, keepdims=True)
    c2 = lax.dot_general(p2, U, (((1,), (0,)), ((), ())),
                         precision=lax.Precision.HIGHEST,
                         preferred_element_type=jnp.float32)

    cnt = jnp.sum((c2 < 0.5).astype(jnp.int32), axis=1, keepdims=True)
    oh = lax.broadcasted_iota(jnp.int32, (B, K), 1) == cnt
    token = jnp.sum(jnp.where(oh, si, 0.0), axis=1, keepdims=True)
    out_ref[:] = token.astype(jnp.int32)


_tc_finish = pl.pallas_call(
    _tc_body,
    out_shape=jax.ShapeDtypeStruct((B, 1), jnp.int32),
)


@jax.jit
def kernel(token_logits, sampling_params):
    cv, ci = _sc_select(token_logits)
    tk = sampling_params[:, 0:1]
    tp = sampling_params[:, 1:2]
    tt = sampling_params[:, 2:3]
    return _tc_finish(cv, ci, tk, tp, tt).reshape(-1)


# speedup vs baseline: 67.9083x; 1.0461x over previous
import functools

import jax
import jax.numpy as jnp
from jax import lax
from jax.experimental import pallas as pl
from jax.experimental.pallas import tpu as pltpu
from jax.experimental.pallas import tpu_sc as plsc

NC, NS, L = 2, 16, 16
NW = NC * NS
B, V = 64, 100000
ROWS_PER_W = B // NW
NVREG = V // L
CAND = 256
NEG = -3.4028235e38
IGNORED = -3000.0


CAND1 = 4096
NB = 1024
UNROLL = 10
NBLK = NVREG // UNROLL


def _monotone(b):
    s = b >> 31
    return b ^ (s & 0x7FFFFFFF)


def _sc_body(logits_hbm, out_vals_hbm, out_idx_hbm, row_v, hist_v,
             cv1_v, ci1_v, cv2_v, ci2_v):
    wid = lax.axis_index("s") * NC + lax.axis_index("c")
    lane = lax.broadcasted_iota(jnp.int32, (L,), 0)
    ones = jnp.ones((L,), jnp.int32)
    zeros16 = jnp.zeros((L,), jnp.int32)
    negs16 = jnp.full((L,), NEG, jnp.float32)
    lanebase_c = lane * (NB + 1) + 512
    lanebase_r = lane * 257

    def scan_hist(nb, rank):
        ngr = nb // L
        stride = nb + 1

        @plsc.parallel_loop(0, ngr, carry=(jnp.int32(0), jnp.int32(0),
                                           jnp.int32(0), jnp.int32(0),
                                           zeros16))
        def gscan(a, carry):
            acc, found, gsel, accsel, tvsel = carry
            g = (ngr - 1) - a
            tv = hist_v[pl.ds(g * L, L)]
            for l in range(1, L):
                tv = tv + hist_v[pl.ds(l * stride + g * L, L)]
            gsum = jnp.sum(tv)
            hit = jnp.logical_and(found == 0, acc + gsum >= rank)
            gsel = jnp.where(hit, g, gsel)
            accsel = jnp.where(hit, acc, accsel)
            tvsel = jnp.where(hit, tv, tvsel)
            found = jnp.where(hit, 1, found)
            acc = jnp.where(found == 0, acc + gsum, acc)
            return acc, found, gsel, accsel, tvsel
        _, _, gsel, accsel, tvsel = gscan
        rtv = lax.rev(tvsel, (0,))
        inc = plsc.cumsum(rtv)
        crossed = (accsel + inc) >= rank
        i0 = jnp.max(plsc.all_reduce_ffs(crossed))
        bucket = gsel * L + (15 - i0)
        above = accsel + jnp.sum(jnp.where(lane == i0, inc - rtv, 0))
        return bucket, rank - above

    def refine_pass(dshift, pshift, pval, rank):
        @plsc.parallel_loop(0, 4112 // L)
        def _(k):
            hist_v[pl.ds(k * L, L)] = zeros16

        @plsc.parallel_loop(0, CAND1 // L, unroll=8)
        def _(i):
            u = _monotone(plsc.bitcast(cv1_v[pl.ds(i * L, L)], jnp.int32))
            if dshift == 24:
                d = (u >> 24) + 128
            else:
                d = (u >> dshift) & 0xFF
            slot = lanebase_r + d
            if pshift is None:
                plsc.addupdate_scatter(hist_v, [slot], ones)
            else:
                m = (u >> pshift) == pval
                plsc.addupdate_scatter(hist_v, [slot], ones, mask=m)

        return scan_hist(256, rank)

    for rr in range(ROWS_PER_W):
        row = wid * ROWS_PER_W + rr
        pltpu.sync_copy(logits_hbm.at[row], row_v)

        @plsc.parallel_loop(0, 16400 // L, unroll=5)
        def _(k):
            hist_v[pl.ds(k * L, L)] = zeros16

        @plsc.parallel_loop(0, NVREG, unroll=UNROLL)
        def _(i):
            u = _monotone(plsc.bitcast(row_v[pl.ds(i * L, L)], jnp.int32))
            plsc.addupdate_scatter(hist_v, [lanebase_c + (u >> 22)], ones)

        b10, _r = scan_hist(NB, jnp.int32(50))
        thresh1 = (b10 - 512) << 22

        @plsc.parallel_loop(0, CAND1 // L, unroll=8)
        def _(k):
            cv1_v[pl.ds(k * L, L)] = negs16

        @plsc.parallel_loop(0, NVREG, unroll=UNROLL, carry=zeros16)
        def c1ptr(i, ptr):
            v = row_v[pl.ds(i * L, L)]
            u = _monotone(plsc.bitcast(v, jnp.int32))
            m = u >= thresh1
            mi = m.astype(jnp.int32)
            offs = ptr + (plsc.cumsum(mi) - mi)
            mw = jnp.logical_and(m, offs < CAND1)
            plsc.store_scatter(cv1_v, [offs], v, mask=mw)
            plsc.store_scatter(ci1_v, [offs], i * L + lane, mask=mw)
            return ptr + plsc.all_reduce_population_count(m)
        del c1ptr

        b0, r1 = refine_pass(24, None, None, jnp.int32(50))
        d0s = b0 - 128
        b1, r2 = refine_pass(16, 24, d0s, r1)
        pre16 = (d0s << 8) | b1
        b2, _ = refine_pass(8, 16, pre16, r2)
        pre24 = (pre16 << 8) | b2
        thresh2 = pre24 << 8

        @plsc.parallel_loop(0, CAND // L, unroll=4)
        def _(k):
            cv2_v[pl.ds(k * L, L)] = negs16
            ci2_v[pl.ds(k * L, L)] = zeros16

        @plsc.parallel_loop(0, CAND1 // L, unroll=8, carry=zeros16)
        def c2ptr(i, ptr):
            v = cv1_v[pl.ds(i * L, L)]
            ivec = ci1_v[pl.ds(i * L, L)]
            u = _monotone(plsc.bitcast(v, jnp.int32))
            m = u >= thresh2
            mi = m.astype(jnp.int32)
            offs = ptr + (plsc.cumsum(mi) - mi)
            mw = jnp.logical_and(m, offs < CAND)
            plsc.store_scatter(cv2_v, [offs], v, mask=mw)
            plsc.store_scatter(ci2_v, [offs], ivec, mask=mw)
            return ptr + plsc.all_reduce_population_count(m)
        del c2ptr

        pltpu.sync_copy(cv2_v, out_vals_hbm.at[row])
        pltpu.sync_copy(ci2_v, out_idx_hbm.at[row])


_sc_select = pl.kernel(
    _sc_body,
    out_type=[jax.ShapeDtypeStruct((B, CAND), jnp.float32),
              jax.ShapeDtypeStruct((B, CAND), jnp.int32)],
    mesh=plsc.VectorSubcoreMesh(core_axis_name="c", subcore_axis_name="s",
                                num_cores=NC, num_subcores=NS),
    scratch_types=[pltpu.VMEM((V,), jnp.float32),
                   pltpu.VMEM((16400,), jnp.int32),
                   pltpu.VMEM((CAND1,), jnp.float32),
                   pltpu.VMEM((CAND1,), jnp.int32),
                   pltpu.VMEM((CAND,), jnp.float32),
                   pltpu.VMEM((CAND,), jnp.int32)],
    compiler_params=pltpu.CompilerParams(needs_layout_passes=False),
)


def _tc_body(vals_ref, idx_ref, tk_ref, tp_ref, tt_ref, out_ref):
    K = 50
    vals = vals_ref[:]
    idxf = idx_ref[:].astype(jnp.float32)

    sv_cols = []
    si_cols = []
    for _ in range(K):
        m = jnp.max(vals, axis=1, keepdims=True)
        is_m = vals == m
        isel = jnp.min(jnp.where(is_m, idxf, jnp.float32(1e9)), axis=1,
                       keepdims=True)
        sv_cols.append(m)
        si_cols.append(isel)
        vals = jnp.where(jnp.logical_and(is_m, idxf == isel), NEG, vals)
    sv = jnp.concatenate(sv_cols, axis=1)
    si = jnp.concatenate(si_cols, axis=1)

    tk = tk_ref[:]
    tp = tp_ref[:]
    tt = tt_ref[:]

    posf = lax.broadcasted_iota(jnp.int32, (B, K), 1).astype(jnp.float32)
    s1 = jnp.where(posf >= tk, IGNORED, sv) / tt

    m1 = jnp.max(s1, axis=1, keepdims=True)
    e1 = jnp.exp(s1 - m1)
    p1 = e1 / jnp.sum(e1, axis=1, keepdims=True)

    rr = lax.broadcasted_iota(jnp.int32, (K, K), 0)
    cc = lax.broadcasted_iota(jnp.int32, (K, K), 1)
    U = (rr <= cc).astype(jnp.float32)
    c1 = lax.dot_general(p1, U, (((1,), (0,)), ((), ())),
                         precision=lax.Precision.HIGHEST,
                         preferred_element_type=jnp.float32)

    gmin = jnp.min(p1[:, 0:1])
    tpe = jnp.maximum(gmin, tp)
    mask2 = jnp.logical_and(c1 > tpe, posf >= 1.0)
    s2 = jnp.where(mask2, IGNORED, s1)

    m2 = jnp.max(s2, axis=1, keepdims=True)
    e2 = jnp.exp(s2 - m2)
    p2 = e2 / jnp.sum(e2, axis=1, keepdims=True)
    c2 = lax.dot_general(p2, U, (((1,), (0,)), ((), ())),
                         precision=lax.Precision.HIGHEST,
                         preferred_element_type=jnp.float32)

    cnt = jnp.sum((c2 < 0.5).astype(jnp.int32), axis=1, keepdims=True)
    oh = lax.broadcasted_iota(jnp.int32, (B, K), 1) == cnt
    token = jnp.sum(jnp.where(oh, si, 0.0), axis=1, keepdims=True)
    out_ref[:] = token.astype(jnp.int32)


_tc_finish = pl.pallas_call(
    _tc_body,
    out_shape=jax.ShapeDtypeStruct((B, 1), jnp.int32),
)


@jax.jit
def kernel(token_logits, sampling_params):
    cv, ci = _sc_select(token_logits)
    tk = sampling_params[:, 0:1]
    tp = sampling_params[:, 1:2]
    tt = sampling_params[:, 2:3]
    return _tc_finish(cv, ci, tk, tp, tt).reshape(-1)


# speedup vs baseline: 71.1779x; 1.0481x over previous
import functools

import jax
import jax.numpy as jnp
from jax import lax
from jax.experimental import pallas as pl
from jax.experimental.pallas import tpu as pltpu
from jax.experimental.pallas import tpu_sc as plsc

NC, NS, L = 2, 16, 16
NW = NC * NS
B, V = 64, 100000
ROWS_PER_W = B // NW
NVREG = V // L
CAND = 64
NEG = -3.4028235e38
IGNORED = -3000.0


CAND1 = 4096
NB = 1024
UNROLL = 10
NBLK = NVREG // UNROLL


def _monotone(b):
    s = b >> 31
    return b ^ (s & 0x7FFFFFFF)


def _sc_body(logits_hbm, out_vals_hbm, out_idx_hbm, row_v, hist_v,
             ci1_v, cv2_v, ci2_v):
    wid = lax.axis_index("s") * NC + lax.axis_index("c")
    lane = lax.broadcasted_iota(jnp.int32, (L,), 0)
    ones = jnp.ones((L,), jnp.int32)
    zeros16 = jnp.zeros((L,), jnp.int32)
    negs16 = jnp.full((L,), NEG, jnp.float32)
    lanebase_c = lane * (NB + 1) + 512
    lanebase_r = lane * 257

    def scan_hist(nb, rank):
        ngr = nb // L
        stride = nb + 1

        @plsc.parallel_loop(0, ngr, carry=(jnp.int32(0), jnp.int32(0),
                                           jnp.int32(0), jnp.int32(0),
                                           zeros16))
        def gscan(a, carry):
            acc, found, gsel, accsel, tvsel = carry
            g = (ngr - 1) - a
            tv = hist_v[pl.ds(g * L, L)]
            for l in range(1, L):
                tv = tv + hist_v[pl.ds(l * stride + g * L, L)]
            gsum = jnp.sum(tv)
            hit = jnp.logical_and(found == 0, acc + gsum >= rank)
            gsel = jnp.where(hit, g, gsel)
            accsel = jnp.where(hit, acc, accsel)
            tvsel = jnp.where(hit, tv, tvsel)
            found = jnp.where(hit, 1, found)
            acc = jnp.where(found == 0, acc + gsum, acc)
            return acc, found, gsel, accsel, tvsel
        _, _, gsel, accsel, tvsel = gscan
        rtv = lax.rev(tvsel, (0,))
        inc = plsc.cumsum(rtv)
        crossed = (accsel + inc) >= rank
        i0 = jnp.max(plsc.all_reduce_ffs(crossed))
        bucket = gsel * L + (15 - i0)
        above = accsel + jnp.sum(jnp.where(lane == i0, inc - rtv, 0))
        return bucket, rank - above

    def refine_pass(row_v, cnt, dshift, pshift, pval, rank):
        @plsc.parallel_loop(0, 4112 // L)
        def _(k):
            hist_v[pl.ds(k * L, L)] = zeros16

        @plsc.parallel_loop(0, CAND1 // L, unroll=8)
        def _(i):
            idx = ci1_v[pl.ds(i * L, L)]
            v = plsc.load_gather(row_v, [idx])
            u = _monotone(plsc.bitcast(v, jnp.int32))
            if dshift == 24:
                d = (u >> 24) + 128
            else:
                d = (u >> dshift) & 0xFF
            slot = lanebase_r + d
            m = (i * L + lane) < cnt
            if pshift is not None:
                m = jnp.logical_and(m, (u >> pshift) == pval)
            plsc.addupdate_scatter(hist_v, [slot], ones, mask=m)

        return scan_hist(256, rank)

    for rr in range(ROWS_PER_W):
        row = wid * ROWS_PER_W + rr
        pltpu.sync_copy(logits_hbm.at[row], row_v)

        @plsc.parallel_loop(0, 16400 // L, unroll=5)
        def _(k):
            hist_v[pl.ds(k * L, L)] = zeros16

        @plsc.parallel_loop(0, NVREG, unroll=UNROLL)
        def _(i):
            u = _monotone(plsc.bitcast(row_v[pl.ds(i * L, L)], jnp.int32))
            plsc.addupdate_scatter(hist_v, [lanebase_c + (u >> 22)], ones)

        b10, _r = scan_hist(NB, jnp.int32(50))
        thresh1 = jnp.maximum((b10 - 512) << 22,
                              jnp.int32(-2147483648 + 0x800000))
        tb = jnp.where(thresh1 >= 0, thresh1, thresh1 ^ 0x7FFFFFFF)
        fthr = plsc.bitcast(zeros16 + tb, jnp.float32)

        @plsc.parallel_loop(0, CAND1 // L, unroll=8)
        def _(k):
            ci1_v[pl.ds(k * L, L)] = zeros16

        @plsc.parallel_loop(0, NVREG, unroll=UNROLL, carry=zeros16)
        def c1ptr(i, ptr):
            v = row_v[pl.ds(i * L, L)]
            m = v >= fthr
            mi = m.astype(jnp.int32)
            offs = ptr + (plsc.cumsum(mi) - mi)
            mw = jnp.logical_and(m, offs < CAND1)
            plsc.store_scatter(ci1_v, [offs], i * L + lane, mask=mw)
            return ptr + plsc.all_reduce_population_count(m)
        cnt = jnp.minimum(c1ptr, CAND1)

        b0, r1 = refine_pass(row_v, cnt, 24, None, None, jnp.int32(50))
        d0s = b0 - 128
        b1, r2 = refine_pass(row_v, cnt, 16, 24, d0s, r1)
        pre16 = (d0s << 8) | b1
        b2, r3 = refine_pass(row_v, cnt, 8, 16, pre16, r2)
        pre24 = (pre16 << 8) | b2
        b3, _ = refine_pass(row_v, cnt, 0, 8, pre24, r3)
        u50 = (pre24 << 8) | b3

        @plsc.parallel_loop(0, CAND // L, unroll=4)
        def _(k):
            cv2_v[pl.ds(k * L, L)] = negs16
            ci2_v[pl.ds(k * L, L)] = zeros16

        @plsc.parallel_loop(0, CAND1 // L, unroll=8, carry=zeros16)
        def c2ptr(i, ptr):
            ivec = ci1_v[pl.ds(i * L, L)]
            v = plsc.load_gather(row_v, [ivec])
            u = _monotone(plsc.bitcast(v, jnp.int32))
            m = jnp.logical_and(u >= u50, (i * L + lane) < cnt)
            mi = m.astype(jnp.int32)
            offs = ptr + (plsc.cumsum(mi) - mi)
            mw = jnp.logical_and(m, offs < CAND)
            plsc.store_scatter(cv2_v, [offs], v, mask=mw)
            plsc.store_scatter(ci2_v, [offs], ivec, mask=mw)
            return ptr + plsc.all_reduce_population_count(m)
        del c2ptr

        pltpu.sync_copy(cv2_v, out_vals_hbm.at[row])
        pltpu.sync_copy(ci2_v, out_idx_hbm.at[row])


_sc_select = pl.kernel(
    _sc_body,
    out_type=[jax.ShapeDtypeStruct((B, CAND), jnp.float32),
              jax.ShapeDtypeStruct((B, CAND), jnp.int32)],
    mesh=plsc.VectorSubcoreMesh(core_axis_name="c", subcore_axis_name="s",
                                num_cores=NC, num_subcores=NS),
    scratch_types=[pltpu.VMEM((V,), jnp.float32),
                   pltpu.VMEM((16400,), jnp.int32),
                   pltpu.VMEM((CAND1,), jnp.int32),
                   pltpu.VMEM((CAND,), jnp.float32),
                   pltpu.VMEM((CAND,), jnp.int32)],
    compiler_params=pltpu.CompilerParams(needs_layout_passes=False),
)


def _tc_body(vals_ref, idx_ref, tk_ref, tp_ref, tt_ref, out_ref):
    K = 50
    vals = vals_ref[:]
    idxf = idx_ref[:].astype(jnp.float32)

    sv_cols = []
    si_cols = []
    for _ in range(K):
        m = jnp.max(vals, axis=1, keepdims=True)
        is_m = vals == m
        isel = jnp.min(jnp.where(is_m, idxf, jnp.float32(1e9)), axis=1,
                       keepdims=True)
        sv_cols.append(m)
        si_cols.append(isel)
        vals = jnp.where(jnp.logical_and(is_m, idxf == isel), NEG, vals)
    sv = jnp.concatenate(sv_cols, axis=1)
    si = jnp.concatenate(si_cols, axis=1)

    tk = tk_ref[:]
    tp = tp_ref[:]
    tt = tt_ref[:]

    posf = lax.broadcasted_iota(jnp.int32, (B, K), 1).astype(jnp.float32)
    s1 = jnp.where(posf >= tk, IGNORED, sv) / tt

    m1 = jnp.max(s1, axis=1, keepdims=True)
    e1 = jnp.exp(s1 - m1)
    p1 = e1 / jnp.sum(e1, axis=1, keepdims=True)

    rr = lax.broadcasted_iota(jnp.int32, (K, K), 0)
    cc = lax.broadcasted_iota(jnp.int32, (K, K), 1)
    U = (rr <= cc).astype(jnp.float32)
    c1 = lax.dot_general(p1, U, (((1,), (0,)), ((), ())),
                         precision=lax.Precision.HIGHEST,
                         preferred_element_type=jnp.float32)

    gmin = jnp.min(p1[:, 0:1])
    tpe = jnp.maximum(gmin, tp)
    mask2 = jnp.logical_and(c1 > tpe, posf >= 1.0)
    s2 = jnp.where(mask2, IGNORED, s1)

    m2 = jnp.max(s2, axis=1, keepdims=True)
    e2 = jnp.exp(s2 - m2)
    p2 = e2 / jnp.sum(e2, axis=1, keepdims=True)
    c2 = lax.dot_general(p2, U, (((1,), (0,)), ((), ())),
                         precision=lax.Precision.HIGHEST,
                         preferred_element_type=jnp.float32)

    cnt = jnp.sum((c2 < 0.5).astype(jnp.int32), axis=1, keepdims=True)
    oh = lax.broadcasted_iota(jnp.int32, (B, K), 1) == cnt
    token = jnp.sum(jnp.where(oh, si, 0.0), axis=1, keepdims=True)
    out_ref[:] = token.astype(jnp.int32)


_tc_finish = pl.pallas_call(
    _tc_body,
    out_shape=jax.ShapeDtypeStruct((B, 1), jnp.int32),
)


@jax.jit
def kernel(token_logits, sampling_params):
    cv, ci = _sc_select(token_logits)
    tk = sampling_params[:, 0:1]
    tp = sampling_params[:, 1:2]
    tt = sampling_params[:, 2:3]
    return _tc_finish(cv, ci, tk, tp, tt).reshape(-1)
